# baseline (speedup 1.0000x reference)
import functools

import jax
import jax.numpy as jnp
from jax import lax
from jax.experimental import pallas as pl
from jax.experimental.pallas import tpu as pltpu
from jax.experimental.pallas import tpu_sc as plsc

_NC = 2
_NS = 16
_G = 128
_HALF = 128


def _spmm_sc(x2, srcw_r, dst_r, npad):
    ns, nbh, two, gg = srcw_r.shape
    g = gg // 2
    nb = nbh * 2
    n = npad
    rpt = n // ns
    full = rpt // g
    rem = rpt - full * g

    mesh = plsc.VectorSubcoreMesh(
        core_axis_name="c", subcore_axis_name="s",
        num_cores=_NC, num_subcores=ns)

    @functools.partial(
        pl.kernel,
        out_type=(jax.ShapeDtypeStruct((n, _HALF), jnp.float32),
                  jax.ShapeDtypeStruct((n, _HALF), jnp.float32)),
        mesh=mesh,
        scratch_types=[
            pltpu.VMEM((3, g, _HALF), jnp.float32),
            pltpu.VMEM((2, g, _HALF), jnp.float32),
            pltpu.VMEM((4, 2, gg), jnp.int32),
            pltpu.VMEM((4, g), jnp.int32),
            pltpu.VMEM_SHARED((n, _HALF), jnp.float32),
            pltpu.SemaphoreType.DMA((3,)),
            pltpu.SemaphoreType.DMA((2,)),
            pltpu.SemaphoreType.DMA((4,)),
            pltpu.SemaphoreType.DMA((4,)),
        ],
    )
    def k(x2_hbm, srcw_hbm, dst_hbm, lo_hbm, hi_hbm,
          gr, sr, ring, dstr, acc_sh, gsem, ssem, isem, dsem):
        c = lax.axis_index("c")
        s = lax.axis_index("s")
        offv = jnp.full((16,), c * n, jnp.int32)
        zero16 = jnp.zeros((16,), jnp.float32)

        def src_slice(jj):
            return ring.at[lax.rem(jj // 2, 4), 0, pl.ds((jj % 2) * g, g)]

        def fetch_pair(r):
            sl = lax.rem(r, 4)
            pltpu.async_copy(srcw_hbm.at[s, r], ring.at[sl], isem.at[sl])

        def wait_pair(r):
            sl = lax.rem(r, 4)
            pltpu.make_async_copy(srcw_hbm.at[s, r], ring.at[sl],
                                  isem.at[sl]).wait()
            for q in range(gg // 16):
                ring[sl, 0, pl.ds(q * 16, 16)] = (
                    ring[sl, 0, pl.ds(q * 16, 16)] + offv)

        def fetch_dst(jj):
            sl = lax.rem(jj, 4)
            pltpu.async_copy(dst_hbm.at[s, jj // 2, pl.ds((jj % 2) * g, g)],
                             dstr.at[sl], dsem.at[sl])

        def wait_dst(jj):
            sl = lax.rem(jj, 4)
            pltpu.make_async_copy(
                dst_hbm.at[s, jj // 2, pl.ds((jj % 2) * g, g)],
                dstr.at[sl], dsem.at[sl]).wait()

        def gather(jj):
            pltpu.async_copy(x2_hbm.at[src_slice(jj)],
                             gr.at[lax.rem(jj, 3)], gsem.at[lax.rem(jj, 3)])

        def gather_wait(jj):
            pltpu.make_async_copy(x2_hbm.at[src_slice(jj)],
                                  gr.at[lax.rem(jj, 3)],
                                  gsem.at[lax.rem(jj, 3)]).wait()

        def scatter(jj):
            pltpu.async_copy(sr.at[lax.rem(jj, 2)],
                             acc_sh.at[dstr.at[lax.rem(jj, 4)]],
                             ssem.at[lax.rem(jj, 2)], add=True)

        def scatter_wait(jj):
            pltpu.make_async_copy(sr.at[lax.rem(jj, 2)],
                                  acc_sh.at[dstr.at[lax.rem(jj, 4)]],
                                  ssem.at[lax.rem(jj, 2)]).wait()

        def scale(jj):
            m3 = lax.rem(jj, 3)
            m2 = lax.rem(jj, 2)
            sl = lax.rem(jj // 2, 4)
            hof = (jj % 2) * g

            @pl.loop(0, g // 16)
            def _(q):
                wv16 = lax.bitcast_convert_type(
                    ring[sl, 1, pl.ds(hof + q * 16, 16)], jnp.float32)
                for l in range(16):
                    wspl = jnp.full((16,), wv16[l], jnp.float32)
                    e = q * 16 + l
                    for jcol in range(_HALF // 16):
                        sr[m2, e, pl.ds(jcol * 16, 16)] = (
                            gr[m3, e, pl.ds(jcol * 16, 16)] * wspl)

        @pl.loop(0, g)
        def _(e):
            for j in range(_HALF // 16):
                sr[0, e, pl.ds(j * 16, 16)] = zero16

        base = s * rpt

        @pl.loop(0, full)
        def _(k2):
            pltpu.sync_copy(sr.at[0], acc_sh.at[pl.ds(base + k2 * g, g)])
        if rem:
            pltpu.sync_copy(sr.at[0, pl.ds(0, rem)],
                            acc_sh.at[pl.ds(base + full * g, rem)])

        plsc.subcore_barrier()

        for r0 in range(3):
            fetch_pair(r0)
        for j0 in range(2):
            fetch_dst(j0)
        for r0 in range(2):
            wait_pair(r0)
        for j0 in range(3):
            gather(j0)

        @pl.loop(0, nb)
        def _(j):
            gather_wait(j)

            @pl.when(j >= 2)
            def _():
                scatter_wait(j - 2)

            @pl.when(j + 2 < nb)
            def _():
                fetch_dst(j + 2)

            wait_dst(j)
            scale(j)
            scatter(j)

            @pl.when(j + 3 < nb)
            def _():
                @pl.when(lax.rem(j, 2) == 1)
                def _():
                    wait_pair((j + 3) // 2)
                gather(j + 3)

            @pl.when((lax.rem(j, 2) == 0) & (j + 6 < nb))
            def _():
                fetch_pair((j + 6) // 2)

        scatter_wait(nb - 2)
        scatter_wait(nb - 1)

        plsc.subcore_barrier()

        def writeout(out_hbm):
            @pl.loop(0, full)
            def _(k2):
                pltpu.sync_copy(acc_sh.at[pl.ds(base + k2 * g, g)],
                                out_hbm.at[pl.ds(base + k2 * g, g)])
            if rem:
                pltpu.sync_copy(acc_sh.at[pl.ds(base + full * g, rem)],
                                out_hbm.at[pl.ds(base + full * g, rem)])

        @pl.when(c == 0)
        def _():
            writeout(lo_hbm)

        @pl.when(c == 1)
        def _():
            writeout(hi_hbm)

    return k(x2, srcw_r, dst_r)


def _dense_body(x_ref, lo_ref, hi_ref, wl_ref, wm_ref, wh_ref, coef_ref,
                b4_ref, o_ref):
    x = x_ref[...]
    h = x - jnp.concatenate([lo_ref[...], hi_ref[...]], axis=1)
    la = jnp.clip(coef_ref[0], 0.0, 1.0)
    lg = jnp.maximum(coef_ref[1], 0.0)
    ha = jnp.clip(coef_ref[2], 0.0, 1.0)
    hg = jnp.maximum(coef_ref[3], 0.0)
    ma = jnp.clip(coef_ref[4], 0.0, 1.0)
    mg = jnp.maximum(coef_ref[5], 0.0)
    a_low = (x - la * h) * lg
    a_high = (ha * h + (1.0 - 2.0 * ha) * x) * hg
    a_mid = (h * h - ma * x) * mg
    acc = jnp.dot(a_low, wl_ref[...], preferred_element_type=jnp.float32)
    acc = acc + jnp.dot(a_high, wh_ref[...], preferred_element_type=jnp.float32)
    acc = acc + jnp.dot(a_mid, wm_ref[...], preferred_element_type=jnp.float32)
    o_ref[...] = acc + jnp.sum(b4_ref[...], axis=0, keepdims=True)


def _dense_tc(x, lx_lo, lx_hi, w_low, w_mid, w_high, coef, b4):
    n, d = x.shape
    bm = 400
    return pl.pallas_call(
        _dense_body,
        grid=(n // bm,),
        in_specs=[
            pl.BlockSpec((bm, d), lambda i: (i, 0)),
            pl.BlockSpec((bm, _HALF), lambda i: (i, 0)),
            pl.BlockSpec((bm, _HALF), lambda i: (i, 0)),
            pl.BlockSpec((d, d), lambda i: (0, 0)),
            pl.BlockSpec((d, d), lambda i: (0, 0)),
            pl.BlockSpec((d, d), lambda i: (0, 0)),
            pl.BlockSpec(memory_space=pltpu.SMEM),
            pl.BlockSpec((4, d), lambda i: (0, 0)),
        ],
        out_specs=pl.BlockSpec((bm, d), lambda i: (i, 0)),
        out_shape=jax.ShapeDtypeStruct((n, d), jnp.float32),
    )(x, lx_lo, lx_hi, w_low, w_mid, w_high, coef, b4)


def kernel(X, edge_weight, W_low, b_low, W_mid, b_mid, W_high, b_high,
           lowalpha, lowgamma, highalpha, highgamma, midalpha, midgamma,
           bias, edge_index):
    n, d = X.shape
    e = edge_weight.shape[0]
    src = edge_index[0]
    dst = edge_index[1]

    npad = -(-n // (_NS * 8)) * (_NS * 8)
    zrows = jnp.zeros((npad - n, _HALF), jnp.float32)
    x2 = jnp.concatenate(
        [X[:, :_HALF], zrows, X[:, _HALF:], zrows], axis=0)

    nb = -(-e // (_NS * _G))
    nb = nb + (nb % 2)
    ep = _NS * nb * _G
    pad = ep - e
    ar = jnp.arange(pad, dtype=jnp.int32) % n
    src_r = jnp.concatenate([src, ar]).reshape(_NS, nb, _G)
    dst_r = jnp.concatenate([dst, ar]).reshape(_NS, nb, _G)
    w_r = jnp.concatenate(
        [edge_weight, jnp.zeros((pad,), jnp.float32)]).reshape(_NS, nb, _G)
    srcw_r = jnp.stack(
        [src_r, jax.lax.bitcast_convert_type(w_r, jnp.int32)], axis=2)

    lx_lo, lx_hi = _spmm_sc(x2, srcw_r, dst_r, npad)

    coef = jnp.concatenate([lowalpha, lowgamma, highalpha, highgamma,
                            midalpha, midgamma, jnp.zeros((2,), jnp.float32)])
    b4 = jnp.stack([b_low, b_mid, b_high, bias])

    return _dense_tc(X, lx_lo, lx_hi, W_low, W_mid, W_high, coef, b4)


# speedup vs baseline: 1.0003x; 1.0003x over previous
import functools

import jax
import jax.numpy as jnp
from jax import lax
from jax.experimental import pallas as pl
from jax.experimental.pallas import tpu as pltpu
from jax.experimental.pallas import tpu_sc as plsc

_NC = 2
_NS = 16
_G = 128
_HALF = 128


def _spmm_sc(x2, srcw_r, dst_r, npad):
    ns, nbh, two, gg = srcw_r.shape
    g = gg // 2
    nb = nbh * 2
    n = npad
    rpt = n // ns
    full = rpt // g
    rem = rpt - full * g

    mesh = plsc.VectorSubcoreMesh(
        core_axis_name="c", subcore_axis_name="s",
        num_cores=_NC, num_subcores=ns)

    @functools.partial(
        pl.kernel,
        out_type=(jax.ShapeDtypeStruct((n, _HALF), jnp.float32),
                  jax.ShapeDtypeStruct((n, _HALF), jnp.float32)),
        mesh=mesh,
        scratch_types=[
            pltpu.VMEM((2, g, _HALF), jnp.float32),
            pltpu.VMEM((2, g, _HALF), jnp.float32),
            pltpu.VMEM((4, 2, gg), jnp.int32),
            pltpu.VMEM((4, g), jnp.int32),
            pltpu.VMEM_SHARED((n, _HALF), jnp.float32),
            pltpu.SemaphoreType.DMA((2,)),
            pltpu.SemaphoreType.DMA((2,)),
            pltpu.SemaphoreType.DMA((4,)),
            pltpu.SemaphoreType.DMA((4,)),
        ],
    )
    def k(x2_hbm, srcw_hbm, dst_hbm, lo_hbm, hi_hbm,
          gr, sr, ring, dstr, acc_sh, gsem, ssem, isem, dsem):
        c = lax.axis_index("c")
        s = lax.axis_index("s")
        offv = jnp.full((16,), c * n, jnp.int32)
        zero16 = jnp.zeros((16,), jnp.float32)

        def src_slice(jj):
            return ring.at[lax.rem(jj // 2, 4), 0, pl.ds((jj % 2) * g, g)]

        def fetch_pair(r):
            sl = lax.rem(r, 4)
            pltpu.async_copy(srcw_hbm.at[s, r], ring.at[sl], isem.at[sl])

        def wait_pair(r):
            sl = lax.rem(r, 4)
            pltpu.make_async_copy(srcw_hbm.at[s, r], ring.at[sl],
                                  isem.at[sl]).wait()
            for q in range(gg // 16):
                ring[sl, 0, pl.ds(q * 16, 16)] = (
                    ring[sl, 0, pl.ds(q * 16, 16)] + offv)

        def fetch_dst(jj):
            sl = lax.rem(jj, 4)
            pltpu.async_copy(dst_hbm.at[s, jj // 2, pl.ds((jj % 2) * g, g)],
                             dstr.at[sl], dsem.at[sl])

        def wait_dst(jj):
            sl = lax.rem(jj, 4)
            pltpu.make_async_copy(
                dst_hbm.at[s, jj // 2, pl.ds((jj % 2) * g, g)],
                dstr.at[sl], dsem.at[sl]).wait()

        def gather(jj):
            pltpu.async_copy(x2_hbm.at[src_slice(jj)],
                             gr.at[lax.rem(jj, 2)], gsem.at[lax.rem(jj, 2)])

        def gather_wait(jj):
            pltpu.make_async_copy(x2_hbm.at[src_slice(jj)],
                                  gr.at[lax.rem(jj, 2)],
                                  gsem.at[lax.rem(jj, 2)]).wait()

        def scatter(jj):
            pltpu.async_copy(sr.at[lax.rem(jj, 2)],
                             acc_sh.at[dstr.at[lax.rem(jj, 4)]],
                             ssem.at[lax.rem(jj, 2)], add=True)

        def scatter_wait(jj):
            pltpu.make_async_copy(sr.at[lax.rem(jj, 2)],
                                  acc_sh.at[dstr.at[lax.rem(jj, 4)]],
                                  ssem.at[lax.rem(jj, 2)]).wait()

        def scale(jj):
            m3 = lax.rem(jj, 2)
            m2 = lax.rem(jj, 2)
            sl = lax.rem(jj // 2, 4)
            hof = (jj % 2) * g

            @pl.loop(0, g // 16)
            def _(q):
                wv16 = lax.bitcast_convert_type(
                    ring[sl, 1, pl.ds(hof + q * 16, 16)], jnp.float32)
                for l in range(16):
                    wspl = jnp.full((16,), wv16[l], jnp.float32)
                    e = q * 16 + l
                    for jcol in range(_HALF // 16):
                        sr[m2, e, pl.ds(jcol * 16, 16)] = (
                            gr[m3, e, pl.ds(jcol * 16, 16)] * wspl)

        @pl.loop(0, g)
        def _(e):
            for j in range(_HALF // 16):
                sr[0, e, pl.ds(j * 16, 16)] = zero16

        base = s * rpt

        @pl.loop(0, full)
        def _(k2):
            pltpu.sync_copy(sr.at[0], acc_sh.at[pl.ds(base + k2 * g, g)])
        if rem:
            pltpu.sync_copy(sr.at[0, pl.ds(0, rem)],
                            acc_sh.at[pl.ds(base + full * g, rem)])

        plsc.subcore_barrier()

        for r0 in range(4):
            fetch_pair(r0)
        for j0 in range(2):
            fetch_dst(j0)
        wait_pair(0)
        for j0 in range(2):
            gather(j0)

        @pl.loop(0, nb)
        def _(j):
            gather_wait(j)

            @pl.when(j >= 2)
            def _():
                pltpu.make_async_copy(sr.at[lax.rem(j, 2)],
                                      acc_sh.at[dstr.at[lax.rem(j + 2, 4)]],
                                      ssem.at[lax.rem(j, 2)]).wait()

            @pl.when(j + 2 < nb)
            def _():
                fetch_dst(j + 2)

            wait_dst(j)
            scale(j)
            scatter(j)

            @pl.when(j + 2 < nb)
            def _():
                @pl.when(lax.rem(j, 2) == 0)
                def _():
                    wait_pair((j + 2) // 2)
                gather(j + 2)

            @pl.when((lax.rem(j, 2) == 1) & (j + 7 < nb))
            def _():
                fetch_pair((j + 7) // 2)

        scatter_wait(nb - 2)
        scatter_wait(nb - 1)

        plsc.subcore_barrier()

        def writeout(out_hbm):
            @pl.loop(0, full)
            def _(k2):
                pltpu.sync_copy(acc_sh.at[pl.ds(base + k2 * g, g)],
                                out_hbm.at[pl.ds(base + k2 * g, g)])
            if rem:
                pltpu.sync_copy(acc_sh.at[pl.ds(base + full * g, rem)],
                                out_hbm.at[pl.ds(base + full * g, rem)])

        @pl.when(c == 0)
        def _():
            writeout(lo_hbm)

        @pl.when(c == 1)
        def _():
            writeout(hi_hbm)

    return k(x2, srcw_r, dst_r)


def _dense_body(x_ref, lo_ref, hi_ref, wl_ref, wm_ref, wh_ref, coef_ref,
                b4_ref, o_ref):
    x = x_ref[...]
    h = x - jnp.concatenate([lo_ref[...], hi_ref[...]], axis=1)
    la = jnp.clip(coef_ref[0], 0.0, 1.0)
    lg = jnp.maximum(coef_ref[1], 0.0)
    ha = jnp.clip(coef_ref[2], 0.0, 1.0)
    hg = jnp.maximum(coef_ref[3], 0.0)
    ma = jnp.clip(coef_ref[4], 0.0, 1.0)
    mg = jnp.maximum(coef_ref[5], 0.0)
    a_low = (x - la * h) * lg
    a_high = (ha * h + (1.0 - 2.0 * ha) * x) * hg
    a_mid = (h * h - ma * x) * mg
    acc = jnp.dot(a_low, wl_ref[...], preferred_element_type=jnp.float32)
    acc = acc + jnp.dot(a_high, wh_ref[...], preferred_element_type=jnp.float32)
    acc = acc + jnp.dot(a_mid, wm_ref[...], preferred_element_type=jnp.float32)
    o_ref[...] = acc + jnp.sum(b4_ref[...], axis=0, keepdims=True)


def _dense_tc(x, lx_lo, lx_hi, w_low, w_mid, w_high, coef, b4):
    n, d = x.shape
    bm = 400
    return pl.pallas_call(
        _dense_body,
        grid=(n // bm,),
        in_specs=[
            pl.BlockSpec((bm, d), lambda i: (i, 0)),
            pl.BlockSpec((bm, _HALF), lambda i: (i, 0)),
            pl.BlockSpec((bm, _HALF), lambda i: (i, 0)),
            pl.BlockSpec((d, d), lambda i: (0, 0)),
            pl.BlockSpec((d, d), lambda i: (0, 0)),
            pl.BlockSpec((d, d), lambda i: (0, 0)),
            pl.BlockSpec(memory_space=pltpu.SMEM),
            pl.BlockSpec((4, d), lambda i: (0, 0)),
        ],
        out_specs=pl.BlockSpec((bm, d), lambda i: (i, 0)),
        out_shape=jax.ShapeDtypeStruct((n, d), jnp.float32),
    )(x, lx_lo, lx_hi, w_low, w_mid, w_high, coef, b4)


def kernel(X, edge_weight, W_low, b_low, W_mid, b_mid, W_high, b_high,
           lowalpha, lowgamma, highalpha, highgamma, midalpha, midgamma,
           bias, edge_index):
    n, d = X.shape
    e = edge_weight.shape[0]
    src = edge_index[0]
    dst = edge_index[1]

    npad = -(-n // (_NS * 8)) * (_NS * 8)
    zrows = jnp.zeros((npad - n, _HALF), jnp.float32)
    x2 = jnp.concatenate(
        [X[:, :_HALF], zrows, X[:, _HALF:], zrows], axis=0)

    nb = -(-e // (_NS * _G))
    nb = nb + (nb % 2)
    ep = _NS * nb * _G
    pad = ep - e
    ar = jnp.arange(pad, dtype=jnp.int32) % n
    src_r = jnp.concatenate([src, ar]).reshape(_NS, nb, _G)
    dst_r = jnp.concatenate([dst, ar]).reshape(_NS, nb, _G)
    w_r = jnp.concatenate(
        [edge_weight, jnp.zeros((pad,), jnp.float32)]).reshape(_NS, nb, _G)
    srcw_r = jnp.stack(
        [src_r, jax.lax.bitcast_convert_type(w_r, jnp.int32)], axis=2)

    lx_lo, lx_hi = _spmm_sc(x2, srcw_r, dst_r, npad)

    coef = jnp.concatenate([lowalpha, lowgamma, highalpha, highgamma,
                            midalpha, midgamma, jnp.zeros((2,), jnp.float32)])
    b4 = jnp.stack([b_low, b_mid, b_high, bias])

    return _dense_tc(X, lx_lo, lx_hi, W_low, W_mid, W_high, coef, b4)


# speedup vs baseline: 2.1042x; 2.1035x over previous
import functools

import jax
import jax.numpy as jnp
from jax import lax
from jax.experimental import pallas as pl
from jax.experimental.pallas import tpu as pltpu
from jax.experimental.pallas import tpu_sc as plsc

_NC = 2
_NS = 16
_G = 128
_HALF = 128


def _spmm_sc(x2, src_r, dst_r, w_r, npad):
    ns, nbh, gg = src_r.shape
    g = gg // 2
    nb = nbh * 2
    n = npad
    rpt = n // ns
    full = rpt // g
    rem = rpt - full * g

    mesh = plsc.VectorSubcoreMesh(
        core_axis_name="c", subcore_axis_name="s",
        num_cores=_NC, num_subcores=ns)

    @functools.partial(
        pl.kernel,
        out_type=(jax.ShapeDtypeStruct((n, _HALF), jnp.float32),
                  jax.ShapeDtypeStruct((n, _HALF), jnp.float32)),
        mesh=mesh,
        scratch_types=[
            pltpu.VMEM((g, _HALF), jnp.float32),
            pltpu.VMEM((g, _HALF), jnp.float32),
            pltpu.VMEM((g, _HALF), jnp.float32),
            pltpu.VMEM((g, _HALF), jnp.float32),
            pltpu.VMEM((8, g), jnp.int32),
            pltpu.VMEM((8, g), jnp.int32),
            pltpu.VMEM((8, g), jnp.float32),
            pltpu.VMEM_SHARED((n, _HALF), jnp.float32),
            pltpu.SemaphoreType.DMA,
            pltpu.SemaphoreType.DMA,
            pltpu.SemaphoreType.DMA,
            pltpu.SemaphoreType.DMA,
            pltpu.SemaphoreType.DMA((8,)),
        ],
    )
    def k(x2_hbm, src_hbm, dst_hbm, w_hbm, lo_hbm, hi_hbm,
          gb0, gb1, sb0, sb1, srcr, dstr, wr, acc_sh,
          gsem0, gsem1, ssem0, ssem1, isem):
        c = lax.axis_index("c")
        s = lax.axis_index("s")
        gbufs = (gb0, gb1)
        sbufs = (sb0, sb1)
        gsems = (gsem0, gsem1)
        ssems = (ssem0, ssem1)
        offv = jnp.full((16,), c * n, jnp.int32)
        zero16 = jnp.zeros((16,), jnp.float32)

        def fetch(row, half, sl):
            pltpu.async_copy(src_hbm.at[s, row, pl.ds(half * g, g)],
                             srcr.at[sl], isem.at[sl])
            pltpu.async_copy(dst_hbm.at[s, row, pl.ds(half * g, g)],
                             dstr.at[sl], isem.at[sl])
            pltpu.async_copy(w_hbm.at[s, row, pl.ds(half * g, g)],
                             wr.at[sl], isem.at[sl])

        def fetch_wait(row, half, sl):
            pltpu.make_async_copy(src_hbm.at[s, row, pl.ds(half * g, g)],
                                  srcr.at[sl], isem.at[sl]).wait()
            pltpu.make_async_copy(dst_hbm.at[s, row, pl.ds(half * g, g)],
                                  dstr.at[sl], isem.at[sl]).wait()
            pltpu.make_async_copy(w_hbm.at[s, row, pl.ds(half * g, g)],
                                  wr.at[sl], isem.at[sl]).wait()

        def src_offset(sl):
            for q in range(g // 16):
                srcr[sl, pl.ds(q * 16, 16)] = (
                    srcr[sl, pl.ds(q * 16, 16)] + offv)

        def scale(gb, sb, sl):
            @pl.loop(0, g // 16)
            def _(q):
                wv16 = wr[sl, pl.ds(q * 16, 16)]
                for l in range(16):
                    wspl = jnp.full((16,), wv16[l], jnp.float32)
                    e = q * 16 + l
                    for jj in range(_HALF // 16):
                        sb[e, pl.ds(jj * 16, 16)] = (
                            gb[e, pl.ds(jj * 16, 16)] * wspl)

        for jp in range(4):
            fetch(jp // 2, jp % 2, jp)
        for jp in range(2):
            fetch_wait(jp // 2, jp % 2, jp)
            src_offset(jp)
            pltpu.async_copy(x2_hbm.at[srcr.at[jp]], gbufs[jp], gsems[jp])

        for sb in sbufs:
            @pl.loop(0, g)
            def _(e, sb=sb):
                for j in range(_HALF // 16):
                    sb[e, pl.ds(j * 16, 16)] = zero16

        base = s * rpt

        @pl.loop(0, full)
        def _(k2):
            pltpu.async_copy(sb0, acc_sh.at[pl.ds(base + k2 * g, g)],
                             ssem0)

        @pl.loop(0, full)
        def _(k2):
            pltpu.make_async_copy(sb0, acc_sh.at[pl.ds(base + k2 * g, g)],
                                  ssem0).wait()
        if rem:
            pltpu.sync_copy(sb0.at[pl.ds(0, rem)],
                            acc_sh.at[pl.ds(base + full * g, rem)])

        plsc.subcore_barrier()

        for jp in range(2):
            pltpu.async_copy(sbufs[jp], acc_sh.at[dstr.at[jp]], ssems[jp],
                             add=True)

        @pl.loop(0, nb // 2)
        def _(i):
            for b in range(2):
                j = 2 * i + b
                gb, sb = gbufs[b], sbufs[b]
                sl = lax.rem(j, 8)
                pltpu.make_async_copy(x2_hbm.at[srcr.at[sl]], gb,
                                      gsems[b]).wait()
                pltpu.make_async_copy(sb, acc_sh.at[dstr.at[lax.rem(j + 6, 8)]],
                                      ssems[b]).wait()
                scale(gb, sb, sl)
                pltpu.async_copy(sb, acc_sh.at[dstr.at[sl]], ssems[b],
                                 add=True)

                @pl.when(j + 2 < nb)
                def _(gb=gb, b=b, j=j, i=i):
                    sl2 = lax.rem(j + 2, 8)
                    fetch_wait(i + 1, b, sl2)
                    src_offset(sl2)
                    pltpu.async_copy(x2_hbm.at[srcr.at[sl2]], gb, gsems[b])

                @pl.when(j + 4 < nb)
                def _(b=b, j=j, i=i):
                    fetch(i + 2, b, lax.rem(j + 4, 8))

        for b in range(2):
            pltpu.make_async_copy(sbufs[b],
                                  acc_sh.at[dstr.at[lax.rem(nb - 2 + b, 8)]],
                                  ssems[b]).wait()

        plsc.subcore_barrier()

        def writeout(out_hbm):
            @pl.loop(0, full)
            def _(k2):
                pltpu.async_copy(acc_sh.at[pl.ds(base + k2 * g, g)],
                                 out_hbm.at[pl.ds(base + k2 * g, g)], ssem0)
            if rem:
                pltpu.async_copy(acc_sh.at[pl.ds(base + full * g, rem)],
                                 out_hbm.at[pl.ds(base + full * g, rem)],
                                 ssem1)

            @pl.loop(0, full)
            def _(k2):
                pltpu.make_async_copy(acc_sh.at[pl.ds(base + k2 * g, g)],
                                      out_hbm.at[pl.ds(base + k2 * g, g)],
                                      ssem0).wait()
            if rem:
                pltpu.make_async_copy(
                    acc_sh.at[pl.ds(base + full * g, rem)],
                    out_hbm.at[pl.ds(base + full * g, rem)], ssem1).wait()

        @pl.when(c == 0)
        def _():
            writeout(lo_hbm)

        @pl.when(c == 1)
        def _():
            writeout(hi_hbm)

    return k(x2, src_r, dst_r, w_r)


def _dense_a_body(x_ref, wl_ref, wm_ref, wh_ref, coef_ref, b4_ref,
                  part_ref, who_ref, wh2o_ref):
    la = jnp.clip(coef_ref[0], 0.0, 1.0)
    lg = jnp.maximum(coef_ref[1], 0.0)
    ha = jnp.clip(coef_ref[2], 0.0, 1.0)
    hg = jnp.maximum(coef_ref[3], 0.0)
    ma = jnp.clip(coef_ref[4], 0.0, 1.0)
    mg = jnp.maximum(coef_ref[5], 0.0)
    wl = wl_ref[...]
    wm = wm_ref[...]
    wh = wh_ref[...]
    wx = lg * wl + (hg * (1.0 - 2.0 * ha)) * wh - (mg * ma) * wm
    acc = jnp.dot(x_ref[...], wx, preferred_element_type=jnp.float32)
    part_ref[...] = acc + jnp.sum(b4_ref[...], axis=0, keepdims=True)

    @pl.when(pl.program_id(0) == 0)
    def _():
        who_ref[...] = (hg * ha) * wh - (lg * la) * wl
        wh2o_ref[...] = mg * wm


def _dense_a(x, w_low, w_mid, w_high, coef, b4):
    n, d = x.shape
    bm = 400
    return pl.pallas_call(
        _dense_a_body,
        grid=(n // bm,),
        in_specs=[
            pl.BlockSpec((bm, d), lambda i: (i, 0)),
            pl.BlockSpec((d, d), lambda i: (0, 0)),
            pl.BlockSpec((d, d), lambda i: (0, 0)),
            pl.BlockSpec((d, d), lambda i: (0, 0)),
            pl.BlockSpec(memory_space=pltpu.SMEM),
            pl.BlockSpec((4, d), lambda i: (0, 0)),
        ],
        out_specs=[
            pl.BlockSpec((bm, d), lambda i: (i, 0)),
            pl.BlockSpec((d, d), lambda i: (0, 0)),
            pl.BlockSpec((d, d), lambda i: (0, 0)),
        ],
        out_shape=[
            jax.ShapeDtypeStruct((n, d), jnp.float32),
            jax.ShapeDtypeStruct((d, d), jnp.float32),
            jax.ShapeDtypeStruct((d, d), jnp.float32),
        ],
    )(x, w_low, w_mid, w_high, coef, b4)


def _dense_b_body(part_ref, x_ref, lo_ref, hi_ref, wh_ref, wh2_ref, o_ref):
    x = x_ref[...]
    h = x - jnp.concatenate([lo_ref[...], hi_ref[...]], axis=1)
    acc = part_ref[...]
    acc = acc + jnp.dot(h, wh_ref[...], preferred_element_type=jnp.float32)
    acc = acc + jnp.dot(h * h, wh2_ref[...],
                        preferred_element_type=jnp.float32)
    o_ref[...] = acc


def _dense_b(part, x, lx_lo, lx_hi, wh, wh2):
    n, d = x.shape
    bm = 400
    return pl.pallas_call(
        _dense_b_body,
        grid=(n // bm,),
        in_specs=[
            pl.BlockSpec((bm, d), lambda i: (i, 0)),
            pl.BlockSpec((bm, d), lambda i: (i, 0)),
            pl.BlockSpec((bm, _HALF), lambda i: (i, 0)),
            pl.BlockSpec((bm, _HALF), lambda i: (i, 0)),
            pl.BlockSpec((d, d), lambda i: (0, 0)),
            pl.BlockSpec((d, d), lambda i: (0, 0)),
        ],
        out_specs=pl.BlockSpec((bm, d), lambda i: (i, 0)),
        out_shape=jax.ShapeDtypeStruct((n, d), jnp.float32),
    )(part, x, lx_lo, lx_hi, wh, wh2)


def kernel(X, edge_weight, W_low, b_low, W_mid, b_mid, W_high, b_high,
           lowalpha, lowgamma, highalpha, highgamma, midalpha, midgamma,
           bias, edge_index):
    n, d = X.shape
    e = edge_weight.shape[0]
    src = edge_index[0]
    dst = edge_index[1]

    npad = -(-n // (_NS * 8)) * (_NS * 8)
    zrows = jnp.zeros((npad - n, _HALF), jnp.float32)
    x2 = jnp.concatenate(
        [X[:, :_HALF], zrows, X[:, _HALF:], zrows], axis=0)

    nb = -(-e // (_NS * _G))
    nb = nb + (nb % 2)
    ep = _NS * nb * _G
    pad = ep - e
    ar = jnp.arange(pad, dtype=jnp.int32) % n
    src_r = jnp.concatenate([src, ar]).reshape(_NS, nb, _G)
    dst_r = jnp.concatenate([dst, ar]).reshape(_NS, nb, _G)
    w_r = jnp.concatenate(
        [edge_weight, jnp.zeros((pad,), jnp.float32)]).reshape(_NS, nb, _G)

    lx_lo, lx_hi = _spmm_sc(x2, src_r, dst_r, w_r, npad)

    coef = jnp.concatenate([lowalpha, lowgamma, highalpha, highgamma,
                            midalpha, midgamma, jnp.zeros((2,), jnp.float32)])
    b4 = jnp.stack([b_low, b_mid, b_high, bias])

    part, wh, wh2 = _dense_a(X, W_low, W_mid, W_high, coef, b4)
    return _dense_b(part, X, lx_lo, lx_hi, wh, wh2)


# speedup vs baseline: 2.1141x; 1.0047x over previous
import functools

import jax
import jax.numpy as jnp
from jax import lax
from jax.experimental import pallas as pl
from jax.experimental.pallas import tpu as pltpu
from jax.experimental.pallas import tpu_sc as plsc

_NC = 2
_NS = 16
_G = 128
_HALF = 128


def _spmm_sc(x2, src_r, dst_r, w_r, npad):
    ns, nbh, gg = src_r.shape
    g = gg // 2
    nb = nbh * 2
    n = npad
    rpt = n // ns
    full = rpt // g
    rem = rpt - full * g

    mesh = plsc.VectorSubcoreMesh(
        core_axis_name="c", subcore_axis_name="s",
        num_cores=_NC, num_subcores=ns)

    @functools.partial(
        pl.kernel,
        out_type=(jax.ShapeDtypeStruct((n, _HALF), jnp.float32),
                  jax.ShapeDtypeStruct((n, _HALF), jnp.float32)),
        mesh=mesh,
        scratch_types=[
            pltpu.VMEM((g, _HALF), jnp.float32),
            pltpu.VMEM((g, _HALF), jnp.float32),
            pltpu.VMEM((g, _HALF), jnp.float32),
            pltpu.VMEM((g, _HALF), jnp.float32),
            pltpu.VMEM((8, g), jnp.int32),
            pltpu.VMEM((8, g), jnp.int32),
            pltpu.VMEM((8, g), jnp.float32),
            pltpu.VMEM_SHARED((n, _HALF), jnp.float32),
            pltpu.SemaphoreType.DMA,
            pltpu.SemaphoreType.DMA,
            pltpu.SemaphoreType.DMA,
            pltpu.SemaphoreType.DMA,
            pltpu.SemaphoreType.DMA((8,)),
        ],
    )
    def k(x2_hbm, src_hbm, dst_hbm, w_hbm, lo_hbm, hi_hbm,
          gb0, gb1, sb0, sb1, srcr, dstr, wr, acc_sh,
          gsem0, gsem1, ssem0, ssem1, isem):
        c = lax.axis_index("c")
        s = lax.axis_index("s")
        gbufs = (gb0, gb1)
        sbufs = (sb0, sb1)
        gsems = (gsem0, gsem1)
        ssems = (ssem0, ssem1)
        offv = jnp.full((16,), c * n, jnp.int32)
        zero16 = jnp.zeros((16,), jnp.float32)

        def fetch(row, half, sl):
            pltpu.async_copy(src_hbm.at[s, row, pl.ds(half * g, g)],
                             srcr.at[sl], isem.at[sl])
            pltpu.async_copy(dst_hbm.at[s, row, pl.ds(half * g, g)],
                             dstr.at[sl], isem.at[sl])
            pltpu.async_copy(w_hbm.at[s, row, pl.ds(half * g, g)],
                             wr.at[sl], isem.at[sl])

        def fetch_wait(row, half, sl):
            pltpu.make_async_copy(src_hbm.at[s, row, pl.ds(half * g, g)],
                                  srcr.at[sl], isem.at[sl]).wait()
            pltpu.make_async_copy(dst_hbm.at[s, row, pl.ds(half * g, g)],
                                  dstr.at[sl], isem.at[sl]).wait()
            pltpu.make_async_copy(w_hbm.at[s, row, pl.ds(half * g, g)],
                                  wr.at[sl], isem.at[sl]).wait()

        def src_offset(sl):
            for q in range(g // 16):
                srcr[sl, pl.ds(q * 16, 16)] = (
                    srcr[sl, pl.ds(q * 16, 16)] + offv)

        def scale(gb, sb, sl):
            @pl.loop(0, g // 16)
            def _(q):
                wv16 = wr[sl, pl.ds(q * 16, 16)]
                for l in range(16):
                    wspl = jnp.full((16,), wv16[l], jnp.float32)
                    e = q * 16 + l
                    for jj in range(_HALF // 16):
                        sb[e, pl.ds(jj * 16, 16)] = (
                            gb[e, pl.ds(jj * 16, 16)] * wspl)

        for sb in sbufs:
            @pl.loop(0, g)
            def _(e, sb=sb):
                for j in range(_HALF // 16):
                    sb[e, pl.ds(j * 16, 16)] = zero16

        base = s * rpt

        @pl.loop(0, full)
        def _(k2):
            pltpu.sync_copy(sb0, acc_sh.at[pl.ds(base + k2 * g, g)])
        if rem:
            pltpu.sync_copy(sb0.at[pl.ds(0, rem)],
                            acc_sh.at[pl.ds(base + full * g, rem)])

        plsc.subcore_barrier()

        for jp in range(4):
            fetch(jp // 2, jp % 2, jp)
        for jp in range(2):
            fetch_wait(jp // 2, jp % 2, jp)
            src_offset(jp)
            pltpu.async_copy(x2_hbm.at[srcr.at[jp]], gbufs[jp], gsems[jp])
            pltpu.async_copy(sbufs[jp], acc_sh.at[dstr.at[jp]], ssems[jp],
                             add=True)

        @pl.loop(0, nb // 2)
        def _(i):
            for b in range(2):
                j = 2 * i + b
                gb, sb = gbufs[b], sbufs[b]
                sl = lax.rem(j, 8)
                pltpu.make_async_copy(x2_hbm.at[srcr.at[sl]], gb,
                                      gsems[b]).wait()
                pltpu.make_async_copy(sb, acc_sh.at[dstr.at[lax.rem(j + 6, 8)]],
                                      ssems[b]).wait()
                scale(gb, sb, sl)
                pltpu.async_copy(sb, acc_sh.at[dstr.at[sl]], ssems[b],
                                 add=True)

                @pl.when(j + 2 < nb)
                def _(gb=gb, b=b, j=j, i=i):
                    sl2 = lax.rem(j + 2, 8)
                    fetch_wait(i + 1, b, sl2)
                    src_offset(sl2)
                    pltpu.async_copy(x2_hbm.at[srcr.at[sl2]], gb, gsems[b])

                @pl.when(j + 4 < nb)
                def _(b=b, j=j, i=i):
                    fetch(i + 2, b, lax.rem(j + 4, 8))

        for b in range(2):
            pltpu.make_async_copy(sbufs[b],
                                  acc_sh.at[dstr.at[lax.rem(nb - 2 + b, 8)]],
                                  ssems[b]).wait()

        plsc.subcore_barrier()

        def writeout(out_hbm):
            @pl.loop(0, full)
            def _(k2):
                pltpu.sync_copy(acc_sh.at[pl.ds(base + k2 * g, g)],
                                out_hbm.at[pl.ds(base + k2 * g, g)])
            if rem:
                pltpu.sync_copy(acc_sh.at[pl.ds(base + full * g, rem)],
                                out_hbm.at[pl.ds(base + full * g, rem)])

        @pl.when(c == 0)
        def _():
            writeout(lo_hbm)

        @pl.when(c == 1)
        def _():
            writeout(hi_hbm)

    return k(x2, src_r, dst_r, w_r)


def _dense_body(x_ref, lo_ref, hi_ref, wl_ref, wm_ref, wh_ref, coef_ref,
                b4_ref, o_ref):
    x = x_ref[...]
    h = x - jnp.concatenate([lo_ref[...], hi_ref[...]], axis=1)
    la = jnp.clip(coef_ref[0], 0.0, 1.0)
    lg = jnp.maximum(coef_ref[1], 0.0)
    ha = jnp.clip(coef_ref[2], 0.0, 1.0)
    hg = jnp.maximum(coef_ref[3], 0.0)
    ma = jnp.clip(coef_ref[4], 0.0, 1.0)
    mg = jnp.maximum(coef_ref[5], 0.0)
    a_low = (x - la * h) * lg
    a_high = (ha * h + (1.0 - 2.0 * ha) * x) * hg
    a_mid = (h * h - ma * x) * mg
    acc = jnp.dot(a_low, wl_ref[...], preferred_element_type=jnp.float32)
    acc = acc + jnp.dot(a_high, wh_ref[...], preferred_element_type=jnp.float32)
    acc = acc + jnp.dot(a_mid, wm_ref[...], preferred_element_type=jnp.float32)
    o_ref[...] = acc + jnp.sum(b4_ref[...], axis=0, keepdims=True)


def _dense_tc(x, lx_lo, lx_hi, w_low, w_mid, w_high, coef, b4):
    n, d = x.shape
    bm = 400
    return pl.pallas_call(
        _dense_body,
        grid=(n // bm,),
        in_specs=[
            pl.BlockSpec((bm, d), lambda i: (i, 0)),
            pl.BlockSpec((bm, _HALF), lambda i: (i, 0)),
            pl.BlockSpec((bm, _HALF), lambda i: (i, 0)),
            pl.BlockSpec((d, d), lambda i: (0, 0)),
            pl.BlockSpec((d, d), lambda i: (0, 0)),
            pl.BlockSpec((d, d), lambda i: (0, 0)),
            pl.BlockSpec(memory_space=pltpu.SMEM),
            pl.BlockSpec((4, d), lambda i: (0, 0)),
        ],
        out_specs=pl.BlockSpec((bm, d), lambda i: (i, 0)),
        out_shape=jax.ShapeDtypeStruct((n, d), jnp.float32),
    )(x, lx_lo, lx_hi, w_low, w_mid, w_high, coef, b4)


def kernel(X, edge_weight, W_low, b_low, W_mid, b_mid, W_high, b_high,
           lowalpha, lowgamma, highalpha, highgamma, midalpha, midgamma,
           bias, edge_index):
    n, d = X.shape
    e = edge_weight.shape[0]
    src = edge_index[0]
    dst = edge_index[1]

    npad = -(-n // (_NS * 8)) * (_NS * 8)
    zrows = jnp.zeros((npad - n, _HALF), jnp.float32)
    x2 = jnp.concatenate(
        [X[:, :_HALF], zrows, X[:, _HALF:], zrows], axis=0)

    nb = -(-e // (_NS * _G))
    nb = nb + (nb % 2)
    ep = _NS * nb * _G
    pad = ep - e
    ar = jnp.arange(pad, dtype=jnp.int32) % n
    src_r = jnp.concatenate([src, ar]).reshape(_NS, nb, _G)
    dst_r = jnp.concatenate([dst, ar]).reshape(_NS, nb, _G)
    w_r = jnp.concatenate(
        [edge_weight, jnp.zeros((pad,), jnp.float32)]).reshape(_NS, nb, _G)

    lx_lo, lx_hi = _spmm_sc(x2, src_r, dst_r, w_r, npad)

    coef = jnp.concatenate([lowalpha, lowgamma, highalpha, highgamma,
                            midalpha, midgamma, jnp.zeros((2,), jnp.float32)])
    b4 = jnp.stack([b_low, b_mid, b_high, bias])

    return _dense_tc(X, lx_lo, lx_hi, W_low, W_mid, W_high, coef, b4)
